# baseline (speedup 1.0000x reference)
import functools

import jax
import jax.numpy as jnp
from jax import lax
from jax.experimental import pallas as pl
from jax.experimental.pallas import tpu as pltpu
from jax.experimental.pallas import tpu_sc as plsc

T = 2048
D = 768
F = 3072
E = 8
K = 2
BLOCK_M = 256
NT = (T * K) // BLOCK_M + E
M_PAD = NT * BLOCK_M
NSLOT = T * K

_MESH = plsc.VectorSubcoreMesh(core_axis_name="c", subcore_axis_name="s")


def _disp_body(hs_hbm, srctok_hbm, xpad_hbm, idx_v, rows_v, sem):
    c = lax.axis_index("c")
    s = lax.axis_index("s")
    wid = s * 2 + c
    for half in range(2):
        base = wid * (M_PAD // 32) + half * (M_PAD // 64)
        pltpu.sync_copy(srctok_hbm.at[pl.ds(base, M_PAD // 64)], idx_v)
        pltpu.async_copy(hs_hbm.at[idx_v], rows_v, sem).wait()
        pltpu.sync_copy(rows_v, xpad_hbm.at[pl.ds(base, M_PAD // 64)])


_dispatch = functools.partial(
    pl.kernel,
    out_type=jax.ShapeDtypeStruct((M_PAD, D), jnp.float32),
    mesh=_MESH,
    scratch_types=[
        pltpu.VMEM((M_PAD // 64,), jnp.int32),
        pltpu.VMEM((M_PAD // 64, D), jnp.float32),
        pltpu.SemaphoreType.DMA,
    ],
)(_disp_body)


def _gu_body(eot_ref, rows_ref, x_ref, wg_ref, wu_ref, g_ref):
    m = pl.program_id(0)

    @pl.when(rows_ref[m] > 0)
    def _():
        x = x_ref[...]
        hg = lax.dot_general(x, wg_ref[0], (((1,), (1,)), ((), ())),
                             preferred_element_type=jnp.float32)
        hu = lax.dot_general(x, wu_ref[0], (((1,), (1,)), ((), ())),
                             preferred_element_type=jnp.float32)
        g_ref[...] = hg * jax.nn.sigmoid(hg) * hu


_grouped_gu = pl.pallas_call(
    _gu_body,
    grid_spec=pltpu.PrefetchScalarGridSpec(
        num_scalar_prefetch=2,
        grid=(NT,),
        in_specs=[
            pl.BlockSpec((BLOCK_M, D), lambda m, eot, rows: (m, 0)),
            pl.BlockSpec((1, F, D), lambda m, eot, rows: (eot[m], 0, 0)),
            pl.BlockSpec((1, F, D), lambda m, eot, rows: (eot[m], 1, 0)),
        ],
        out_specs=pl.BlockSpec((BLOCK_M, F), lambda m, eot, rows: (m, 0)),
    ),
    out_shape=jax.ShapeDtypeStruct((M_PAD, F), jnp.float32),
)


def _down_body(eot_ref, rows_ref, g_ref, wd_ref, ws_ref, o_ref):
    m = pl.program_id(0)

    @pl.when(rows_ref[m] > 0)
    def _():
        o = lax.dot_general(g_ref[...], wd_ref[0], (((1,), (1,)), ((), ())),
                            preferred_element_type=jnp.float32)
        o_ref[...] = o * ws_ref[0, 0, :][:, None]


_grouped_down = pl.pallas_call(
    _down_body,
    grid_spec=pltpu.PrefetchScalarGridSpec(
        num_scalar_prefetch=2,
        grid=(NT,),
        in_specs=[
            pl.BlockSpec((BLOCK_M, F), lambda m, eot, rows: (m, 0)),
            pl.BlockSpec((1, D, F), lambda m, eot, rows: (eot[m], 0, 0)),
            pl.BlockSpec((1, 1, BLOCK_M), lambda m, eot, rows: (m, 0, 0)),
        ],
        out_specs=pl.BlockSpec((BLOCK_M, D), lambda m, eot, rows: (m, 0)),
    ),
    out_shape=jax.ShapeDtypeStruct((M_PAD, D), jnp.float32),
)


def _comb_body(y_hbm, pos_hbm, out_hbm, idx_v, rows_v, outbuf, sem):
    c = lax.axis_index("c")
    s = lax.axis_index("s")
    wid = s * 2 + c
    for half in range(2):
        sbase = wid * 128 + half * 64
        pltpu.sync_copy(pos_hbm.at[pl.ds(sbase, 64)], idx_v)
        pltpu.async_copy(y_hbm.at[idx_v], rows_v, sem).wait()

        def tok_body(i, _):
            for k in range(D // 16):
                outbuf[i, pl.ds(16 * k, 16)] = (
                    rows_v[2 * i, pl.ds(16 * k, 16)]
                    + rows_v[2 * i + 1, pl.ds(16 * k, 16)])
            return 0

        lax.fori_loop(0, 32, tok_body, 0)
        pltpu.sync_copy(outbuf, out_hbm.at[pl.ds(wid * 64 + half * 32, 32)])


_combine = functools.partial(
    pl.kernel,
    out_type=jax.ShapeDtypeStruct((T, D), jnp.float32),
    mesh=_MESH,
    scratch_types=[
        pltpu.VMEM((64,), jnp.int32),
        pltpu.VMEM((64, D), jnp.float32),
        pltpu.VMEM((32, D), jnp.float32),
        pltpu.SemaphoreType.DMA,
    ],
)(_comb_body)


def kernel(hidden_states, topk_weights, topk_ids, gate_up_weights, down_weights):
    flat_ids = topk_ids.reshape(-1).astype(jnp.int32)
    flat_w = topk_weights.reshape(-1)
    tok_of_slot = (jnp.arange(T * K, dtype=jnp.int32) // K)

    onehot = (flat_ids[:, None] == jnp.arange(E, dtype=jnp.int32)[None, :])
    csum = jnp.cumsum(onehot.astype(jnp.int32), axis=0)
    counts = csum[-1]
    rank = jnp.sum(jnp.where(onehot, csum - 1, 0), axis=1)

    tiles_per_e = (counts + BLOCK_M - 1) // BLOCK_M
    cum_tiles = jnp.cumsum(tiles_per_e)
    tile_off_e = cum_tiles - tiles_per_e
    pos = tile_off_e[flat_ids] * BLOCK_M + rank

    src_tok = jnp.zeros((M_PAD,), jnp.int32).at[pos].set(tok_of_slot)
    ws = jnp.zeros((M_PAD,), jnp.float32).at[pos].set(flat_w)

    tile_idx = jnp.arange(NT, dtype=jnp.int32)
    eot = jnp.sum(tile_idx[:, None] >= cum_tiles[None, :], axis=1)
    eot = jnp.minimum(eot, E - 1).astype(jnp.int32)
    tile_in_e = tile_idx - tile_off_e[eot]
    rows = jnp.clip(counts[eot] - tile_in_e * BLOCK_M, 0, BLOCK_M).astype(jnp.int32)

    x_pad = _dispatch(hidden_states, src_tok)
    g = _grouped_gu(eot, rows, x_pad, gate_up_weights, gate_up_weights)
    y = _grouped_down(eot, rows, g, down_weights, ws.reshape(NT, 1, BLOCK_M))
    out = _combine(y, pos)
    return out


# speedup vs baseline: 1.3530x; 1.3530x over previous
import functools

import jax
import jax.numpy as jnp
from jax import lax
from jax.experimental import pallas as pl
from jax.experimental.pallas import tpu as pltpu
from jax.experimental.pallas import tpu_sc as plsc

T = 2048
D = 768
F = 3072
E = 8
K = 2
BLOCK_M = 256
NT = (T * K) // BLOCK_M + E
M_PAD = NT * BLOCK_M
NSLOT = T * K
NR = NSLOT // 128

_MESH = plsc.VectorSubcoreMesh(core_axis_name="c", subcore_axis_name="s")

_HI = lax.Precision.HIGHEST


def _meta_body(ids_ref, pos_ref, eot_ref, rows_ref):
    f32, i32 = jnp.float32, jnp.int32
    ids = ids_ref[...]
    tri_incl = (lax.broadcasted_iota(i32, (128, 128), 0)
                <= lax.broadcasted_iota(i32, (128, 128), 1)).astype(f32)
    tri_strict = (lax.broadcasted_iota(i32, (NR, NR), 0)
                  < lax.broadcasted_iota(i32, (NR, NR), 1)).astype(f32)
    tri8 = (lax.broadcasted_iota(i32, (E, E), 0)
            <= lax.broadcasted_iota(i32, (E, E), 1)).astype(f32)

    masks, prefs, rowtots = [], [], []
    for e in range(E):
        m = (ids == e).astype(f32)
        p = lax.dot_general(m, tri_incl, (((1,), (0,)), ((), ())),
                            precision=_HI, preferred_element_type=f32)
        masks.append(m)
        prefs.append(p)
        rowtots.append(p[:, 127:128])
    rowtot = jnp.concatenate(rowtots, axis=1)
    excl = lax.dot_general(tri_strict, rowtot, (((0,), (0,)), ((), ())),
                           precision=_HI, preferred_element_type=f32)
    counts = excl[NR - 1:NR, :] + rowtot[NR - 1:NR, :]
    counts_i = counts.astype(i32)
    tiles_i = (counts_i + (BLOCK_M - 1)) >> 8
    tiles_f = tiles_i.astype(f32)
    cumt = lax.dot_general(tiles_f, tri8, (((1,), (0,)), ((), ())),
                           precision=_HI, preferred_element_type=f32)
    off_f = cumt - tiles_f

    pos = jnp.zeros((NR, 128), f32)
    for e in range(E):
        base_e = excl[:, e:e + 1] + off_f[0:1, e:e + 1] * BLOCK_M
        pos = pos + masks[e] * (prefs[e] - 1.0 + base_e)
    pos_ref[...] = pos.astype(i32)

    ti = lax.broadcasted_iota(i32, (1, 128), 1).astype(f32)
    eot = jnp.zeros((1, 128), f32)
    for e in range(E - 1):
        eot = eot + (ti >= cumt[0:1, e:e + 1]).astype(f32)
    eot_i = jnp.minimum(eot.astype(i32), E - 1)
    rows = jnp.zeros((1, 128), f32)
    for e in range(E):
        rows_e = counts[0:1, e:e + 1] - (ti - off_f[0:1, e:e + 1]) * BLOCK_M
        rows = rows + (eot_i == e).astype(f32) * rows_e
    eot_ref[...] = eot_i
    rows_ref[...] = jnp.clip(rows.astype(i32), 0, BLOCK_M)


_meta = pl.pallas_call(
    _meta_body,
    out_shape=(
        jax.ShapeDtypeStruct((NR, 128), jnp.int32),
        jax.ShapeDtypeStruct((1, 128), jnp.int32),
        jax.ShapeDtypeStruct((1, 128), jnp.int32),
    ),
)


def _disp_body(hs_hbm, pos_hbm, tok_hbm, w_hbm, xpad_hbm, ws_hbm,
               tokbuf, posbuf, wbuf, rows_v, sem):
    c = lax.axis_index("c")
    s = lax.axis_index("s")
    wid = s * 2 + c
    base = wid * 128
    pltpu.sync_copy(pos_hbm.at[pl.ds(base, 128)], posbuf)
    pltpu.sync_copy(tok_hbm.at[pl.ds(base, 128)], tokbuf)
    pltpu.sync_copy(w_hbm.at[pl.ds(base, 128)], wbuf)
    pltpu.async_copy(hs_hbm.at[tokbuf], rows_v, sem).wait()
    pltpu.sync_copy(rows_v, xpad_hbm.at[posbuf])
    pltpu.sync_copy(wbuf, ws_hbm.at[posbuf])


_dispatch = functools.partial(
    pl.kernel,
    out_type=(
        jax.ShapeDtypeStruct((M_PAD, D), jnp.float32),
        jax.ShapeDtypeStruct((M_PAD,), jnp.float32),
    ),
    mesh=_MESH,
    scratch_types=[
        pltpu.VMEM((128,), jnp.int32),
        pltpu.VMEM((128,), jnp.int32),
        pltpu.VMEM((128,), jnp.float32),
        pltpu.VMEM((128, D), jnp.float32),
        pltpu.SemaphoreType.DMA,
    ],
)(_disp_body)


def _gu_body(eot_ref, rows_ref, x_ref, wg_ref, wu_ref, g_ref):
    m = pl.program_id(0)

    @pl.when(rows_ref[m] > 0)
    def _():
        x = x_ref[...]
        hg = lax.dot_general(x, wg_ref[0], (((1,), (1,)), ((), ())),
                             preferred_element_type=jnp.float32)
        hu = lax.dot_general(x, wu_ref[0], (((1,), (1,)), ((), ())),
                             preferred_element_type=jnp.float32)
        g_ref[...] = hg * jax.nn.sigmoid(hg) * hu


_grouped_gu = pl.pallas_call(
    _gu_body,
    grid_spec=pltpu.PrefetchScalarGridSpec(
        num_scalar_prefetch=2,
        grid=(NT,),
        in_specs=[
            pl.BlockSpec((BLOCK_M, D), lambda m, eot, rows: (m, 0)),
            pl.BlockSpec((1, F, D), lambda m, eot, rows: (eot[m], 0, 0)),
            pl.BlockSpec((1, F, D), lambda m, eot, rows: (eot[m], 1, 0)),
        ],
        out_specs=pl.BlockSpec((BLOCK_M, F), lambda m, eot, rows: (m, 0)),
    ),
    out_shape=jax.ShapeDtypeStruct((M_PAD, F), jnp.float32),
)


def _down_body(eot_ref, rows_ref, g_ref, wd_ref, ws_ref, o_ref):
    m = pl.program_id(0)

    @pl.when(rows_ref[m] > 0)
    def _():
        o = lax.dot_general(g_ref[...], wd_ref[0], (((1,), (1,)), ((), ())),
                            preferred_element_type=jnp.float32)
        o_ref[...] = o * ws_ref[0, 0, :][:, None]


_grouped_down = pl.pallas_call(
    _down_body,
    grid_spec=pltpu.PrefetchScalarGridSpec(
        num_scalar_prefetch=2,
        grid=(NT,),
        in_specs=[
            pl.BlockSpec((BLOCK_M, F), lambda m, eot, rows: (m, 0)),
            pl.BlockSpec((1, D, F), lambda m, eot, rows: (eot[m], 0, 0)),
            pl.BlockSpec((1, 1, BLOCK_M), lambda m, eot, rows: (m, 0, 0)),
        ],
        out_specs=pl.BlockSpec((BLOCK_M, D), lambda m, eot, rows: (m, 0)),
    ),
    out_shape=jax.ShapeDtypeStruct((M_PAD, D), jnp.float32),
)


def _comb_body(y_hbm, pos_hbm, out_hbm, idx_v, rows_v, outbuf, sem):
    c = lax.axis_index("c")
    s = lax.axis_index("s")
    wid = s * 2 + c
    for half in range(2):
        sbase = wid * 128 + half * 64
        pltpu.sync_copy(pos_hbm.at[pl.ds(sbase, 64)], idx_v)
        pltpu.async_copy(y_hbm.at[idx_v], rows_v, sem).wait()

        def tok_body(i, _):
            for k in range(D // 16):
                outbuf[i, pl.ds(16 * k, 16)] = (
                    rows_v[2 * i, pl.ds(16 * k, 16)]
                    + rows_v[2 * i + 1, pl.ds(16 * k, 16)])
            return 0

        lax.fori_loop(0, 32, tok_body, 0)
        pltpu.sync_copy(outbuf, out_hbm.at[pl.ds(wid * 64 + half * 32, 32)])


_combine = functools.partial(
    pl.kernel,
    out_type=jax.ShapeDtypeStruct((T, D), jnp.float32),
    mesh=_MESH,
    scratch_types=[
        pltpu.VMEM((64,), jnp.int32),
        pltpu.VMEM((64, D), jnp.float32),
        pltpu.VMEM((32, D), jnp.float32),
        pltpu.SemaphoreType.DMA,
    ],
)(_comb_body)


def kernel(hidden_states, topk_weights, topk_ids, gate_up_weights, down_weights):
    ids2d = topk_ids.reshape(NR, 128).astype(jnp.int32)
    w_flat = topk_weights.reshape(-1)
    tok_flat = jnp.arange(NSLOT, dtype=jnp.int32) // K

    pos2d, eot2d, rows2d = _meta(ids2d)
    pos = pos2d.reshape(NSLOT)
    eot = eot2d.reshape(128)
    rows = rows2d.reshape(128)

    x_pad, ws = _dispatch(hidden_states, pos, tok_flat, w_flat)
    g = _grouped_gu(eot, rows, x_pad, gate_up_weights, gate_up_weights)
    y = _grouped_down(eot, rows, g, down_weights, ws.reshape(NT, 1, BLOCK_M))
    out = _combine(y, pos)
    return out
